# baseline (speedup 1.0000x reference)
import functools

import jax
import jax.numpy as jnp
from jax import lax
from jax.experimental import pallas as pl
from jax.experimental.pallas import tpu as pltpu
from jax.experimental.pallas import tpu_sc as plsc

W, L, NA, NT = 100, 100, 2, 50
NBOX = W * L * NA
NWORK = 32
CHUNK = 640
NPAD = NWORK * CHUNK
STEPS = CHUNK // 16
GROUPS = 4
NTPAD = 64


def _f2i(x):
    return lax.bitcast_convert_type(x, jnp.int32)


def _i2f(x):
    return lax.bitcast_convert_type(x, jnp.float32)


def _sqrt(v):
    y = _i2f(jnp.int32(0x5F3759DF) - lax.shift_right_logical(_f2i(v), 1))
    for _ in range(3):
        y = y * (1.5 - 0.5 * v * y * y)
    return jnp.where(v <= 0.0, 0.0, v * y)


def _ln(x):
    bits = _f2i(x)
    e = lax.shift_right_logical(bits, 23) - 127
    m = _i2f(jnp.bitwise_or(jnp.bitwise_and(bits, 0x007FFFFF),
                            jnp.int32(0x3F800000)))
    big = m > 1.4142135
    m = jnp.where(big, m * 0.5, m)
    e = e + jnp.where(big, 1, 0)
    z = m - 1.0
    t = z / (2.0 + z)
    t2 = t * t
    p = 2.0 * t * (1.0 + t2 * (0.3333333333 + t2 * (0.2 + t2 * (0.14285714 + t2 * 0.11111111))))
    return e.astype(jnp.float32) * 0.6931471805599453 + p


def _sincos(x):
    t = x * 0.6366197723675814 + 12582912.0
    kf = t - 12582912.0
    q = kf.astype(jnp.int32)
    r = x - kf * 1.5707964
    r = r + kf * 4.371139e-08
    r2 = r * r
    sinp = r * (1.0 + r2 * (-0.16666667 + r2 * (0.008333333 + r2 * (-1.984127e-4))))
    cosp = 1.0 + r2 * (-0.5 + r2 * (0.041666668 + r2 * (-0.0013888889 + r2 * 2.4801587e-5)))
    qm = jnp.bitwise_and(q, 3)
    swap = jnp.bitwise_and(q, 1) == 1
    s_ = jnp.where(swap, cosp, sinp)
    c_ = jnp.where(swap, sinp, cosp)
    s_ = jnp.where(qm >= 2, -s_, s_)
    c_ = jnp.where(jnp.bitwise_and(qm + 1, 2) == 2, -c_, c_)
    return s_, c_


def _sc_body(data_hbm, tgt_hbm, tm_hbm, out_hbm,
             data_v, tgt_v, gx0_v, gx1_v, gy0_v, gy1_v, ga_v, tm_v, acc_v,
             *, chunk):
    cid = lax.axis_index("c")
    sid = lax.axis_index("s")
    wid = sid * 2 + cid
    pltpu.sync_copy(data_hbm.at[wid], data_v)
    pltpu.sync_copy(tgt_hbm, tgt_v)
    pltpu.sync_copy(tm_hbm, tm_v)

    tm = tm_v[...]
    r00, r01, r02, t0 = tm[0], tm[1], tm[2], tm[3]
    r10, r11, r12, t1 = tm[4], tm[5], tm[6], tm[7]

    for j in range(NTPAD // 16):
        sl = pl.ds(j * 16, 16)
        tx = tgt_v[0, sl]
        ty = tgt_v[1, sl]
        tw = tgt_v[4, sl]
        tl = tgt_v[5, sl]
        s_, c_ = _sincos(tgt_v[6, sl])
        ex = jnp.abs(c_) * tl * 0.5 + jnp.abs(s_) * tw * 0.5
        ey = jnp.abs(s_) * tl * 0.5 + jnp.abs(c_) * tw * 0.5
        x0 = tx - ex
        x1 = tx + ex
        y0 = ty - ey
        y1 = ty + ey
        gx0_v[sl] = x0
        gx1_v[sl] = x1
        gy0_v[sl] = y0
        gy1_v[sl] = y1
        ga_v[sl] = (x1 - x0) * (y1 - y0)

    def step(i, acc):
        boxes = []
        for g in range(GROUPS):
            sl = pl.ds(i * (16 * GROUPS) + g * 16, 16)
            lg = data_v[14, sl]
            prob = 1.0 / (1.0 + jnp.exp(-lg))
            wgt = jnp.where(prob > 0.1, _ln(1.0 - prob), 0.0)
            d0 = data_v[0, sl]
            d1 = data_v[1, sl]
            d2 = data_v[2, sl]
            d3 = data_v[3, sl]
            d4 = data_v[4, sl]
            d5 = data_v[5, sl]
            d6 = data_v[6, sl]
            a0 = data_v[7, sl]
            a1 = data_v[8, sl]
            a2 = data_v[9, sl]
            a3 = data_v[10, sl]
            a4 = data_v[11, sl]
            a5 = data_v[12, sl]
            a6 = data_v[13, sl]
            ad = _sqrt(a4 * a4 + a5 * a5)
            bx = d0 * ad + a0
            by = d1 * ad + a1
            bz = d2 * a3 + a2
            dh = jnp.exp(d3) * a3
            dw = jnp.exp(d4) * a4
            dl = jnp.exp(d5) * a5
            s_, c_ = _sincos(d6 + a6)
            cx = r00 * bx + r01 * by + r02 * bz + t0
            cy = r10 * bx + r11 * by + r12 * bz + t1
            ex = (jnp.abs(dl * 0.5 * (r00 * c_ + r01 * s_))
                  + jnp.abs(dw * 0.5 * (r01 * c_ - r00 * s_))
                  + jnp.abs(dh * 0.5 * r02))
            ey = (jnp.abs(dl * 0.5 * (r10 * c_ + r11 * s_))
                  + jnp.abs(dw * 0.5 * (r11 * c_ - r10 * s_))
                  + jnp.abs(dh * 0.5 * r12))
            px0 = cx - ex
            px1 = cx + ex
            py0 = cy - ey
            py1 = cy + ey
            pa = (px1 - px0) * (py1 - py0)
            boxes.append((px0, px1, py0, py1, pa, wgt))
        ssum = [jnp.zeros((16,), jnp.float32) for _ in range(GROUPS)]
        for j in range(NT // 16 + 1):
            tsl = pl.ds(j * 16, 16)
            g0 = gx0_v[tsl]
            g1 = gx1_v[tsl]
            h0 = gy0_v[tsl]
            h1 = gy1_v[tsl]
            gg = ga_v[tsl]
            for lane in range(16):
                n = j * 16 + lane
                if n >= NT:
                    break
                b0, b1, c0, c1, aa = g0[lane], g1[lane], h0[lane], h1[lane], gg[lane]
                for g in range(GROUPS):
                    px0, px1, py0, py1, pa, _ = boxes[g]
                    iw = jnp.maximum(0.0, jnp.minimum(px1, b1) - jnp.maximum(px0, b0))
                    ih = jnp.maximum(0.0, jnp.minimum(py1, c1) - jnp.maximum(py0, c0))
                    inter = iw * ih
                    ssum[g] = ssum[g] + inter / (pa + aa - inter)
        for g in range(GROUPS):
            acc = acc + boxes[g][5] * ssum[g]
        return acc

    acc = lax.fori_loop(0, chunk // 16 // GROUPS, step, jnp.zeros((16,), jnp.float32))
    acc_v[...] = acc
    pltpu.sync_copy(acc_v, out_hbm.at[wid])


def _make_sc_call(chunk):
    mesh = plsc.VectorSubcoreMesh(core_axis_name="c", subcore_axis_name="s")
    return functools.partial(
        pl.kernel,
        out_type=jax.ShapeDtypeStruct((NWORK, 16), jnp.float32),
        mesh=mesh,
        scratch_types=[
            pltpu.VMEM((15, chunk), jnp.float32),
            pltpu.VMEM((7, NTPAD), jnp.float32),
            pltpu.VMEM((NTPAD,), jnp.float32),
            pltpu.VMEM((NTPAD,), jnp.float32),
            pltpu.VMEM((NTPAD,), jnp.float32),
            pltpu.VMEM((NTPAD,), jnp.float32),
            pltpu.VMEM((NTPAD,), jnp.float32),
            pltpu.VMEM((16,), jnp.float32),
            pltpu.VMEM((16,), jnp.float32),
        ],
    )(functools.partial(_sc_body, chunk=chunk))


def kernel(psm, rm, anchor_box, transformation_matrix, target):
    f32 = jnp.float32
    dpad = jnp.zeros((NPAD - NBOX,), f32)
    apad = jnp.ones((NPAD - NBOX,), f32)
    lpad = jnp.full((NPAD - NBOX,), -100.0, f32)
    rows = []
    for c in range(7):
        rows.append(jnp.concatenate([rm[0, c].reshape(-1), rm[0, 7 + c].reshape(-1), dpad]))
    for c in range(7):
        rows.append(jnp.concatenate([anchor_box[:, :, 0, c].reshape(-1).astype(f32),
                                     anchor_box[:, :, 1, c].reshape(-1).astype(f32), apad]))
    rows.append(jnp.concatenate([psm[0, 0].reshape(-1), psm[0, 1].reshape(-1), lpad]))
    data = jnp.stack(rows)
    half = CHUNK // 2
    data = data.reshape(15, 2 * NWORK, half)
    data1 = data[:, :NWORK].transpose(1, 0, 2)
    data2 = data[:, NWORK:].transpose(1, 0, 2)
    tgt = jnp.pad(jnp.transpose(target).astype(f32),
                  ((0, 0), (0, NTPAD - NT)), constant_values=1.0)
    tm = jnp.concatenate([transformation_matrix[0].astype(f32),
                          transformation_matrix[1].astype(f32),
                          jnp.zeros((8,), f32)])
    call = _make_sc_call(half)
    out1 = call(data1, tgt, tm)
    out2 = call(data2, tgt, tm)
    return jnp.sum(out1) + jnp.sum(out2)


# speedup vs baseline: 1.4964x; 1.4964x over previous
import functools

import jax
import jax.numpy as jnp
from jax import lax
from jax.experimental import pallas as pl
from jax.experimental.pallas import tpu as pltpu
from jax.experimental.pallas import tpu_sc as plsc

W, L, NA, NT = 100, 100, 2, 50
NWORK = 32
GROUPS = 4
NTPAD = 64
SC_BOXES = W * L
SC_PAD = 10240
SC_CHUNK = SC_PAD // NWORK


def _tc_body(psm_ref, rm_ref, anc_ref, t_ref, tgt_ref, out_ref):
    def sc(i, j):
        return t_ref[i:i + 1, j:j + 1]

    r00, r01, r02, t0 = sc(0, 0), sc(0, 1), sc(0, 2), sc(0, 3)
    r10, r11, r12, t1 = sc(1, 0), sc(1, 1), sc(1, 2), sc(1, 3)

    tx = tgt_ref[0:1, :]
    ty = tgt_ref[1:2, :]
    tw = tgt_ref[4:5, :]
    tl = tgt_ref[5:6, :]
    tc_, ts_ = jnp.cos(tgt_ref[6:7, :]), jnp.sin(tgt_ref[6:7, :])
    tex = jnp.abs(tc_) * tl * 0.5 + jnp.abs(ts_) * tw * 0.5
    tey = jnp.abs(ts_) * tl * 0.5 + jnp.abs(tc_) * tw * 0.5
    gxmin, gxmax = tx - tex, tx + tex
    gymin, gymax = ty - tey, ty + tey
    garea = (gxmax - gxmin) * (gymax - gymin)

    total = jnp.float32(0.0)
    for a in range(psm_ref.shape[0]):
        lg = psm_ref[a]
        prob = jax.nn.sigmoid(lg)
        wgt = jnp.where(prob > 0.1, jnp.log(1.0 - prob), 0.0)

        d = [rm_ref[a * 7 + c] for c in range(7)]
        an = [anc_ref[a * 7 + c] for c in range(7)]
        ad = jnp.sqrt(an[4] * an[4] + an[5] * an[5])
        bx = d[0] * ad + an[0]
        by = d[1] * ad + an[1]
        bz = d[2] * an[3] + an[2]
        dh = jnp.exp(d[3]) * an[3]
        dw = jnp.exp(d[4]) * an[4]
        dl = jnp.exp(d[5]) * an[5]
        ry = d[6] + an[6]
        c, s = jnp.cos(ry), jnp.sin(ry)

        cx = r00 * bx + r01 * by + r02 * bz + t0
        cy = r10 * bx + r11 * by + r12 * bz + t1
        ex = (jnp.abs(dl * 0.5 * (r00 * c + r01 * s))
              + jnp.abs(dw * 0.5 * (r01 * c - r00 * s))
              + jnp.abs(dh * 0.5 * r02))
        ey = (jnp.abs(dl * 0.5 * (r10 * c + r11 * s))
              + jnp.abs(dw * 0.5 * (r11 * c - r10 * s))
              + jnp.abs(dh * 0.5 * r12))
        pxmin, pxmax = cx - ex, cx + ex
        pymin, pymax = cy - ey, cy + ey
        parea = (pxmax - pxmin) * (pymax - pymin)

        iousum = jnp.zeros((W, L), jnp.float32)
        for n in range(NT):
            xm = gxmin[0:1, n:n + 1]
            xM = gxmax[0:1, n:n + 1]
            ym = gymin[0:1, n:n + 1]
            yM = gymax[0:1, n:n + 1]
            ga = garea[0:1, n:n + 1]
            iw = jnp.maximum(0.0, jnp.minimum(pxmax, xM) - jnp.maximum(pxmin, xm))
            ih = jnp.maximum(0.0, jnp.minimum(pymax, yM) - jnp.maximum(pymin, ym))
            inter = iw * ih
            iousum = iousum + inter / (parea + ga - inter)
        total = total + jnp.sum(wgt * iousum)

    out_ref[0, 0] = total


def _tc_call(psm3, rm3, anc, tmat, tgt):
    return pl.pallas_call(
        _tc_body,
        out_shape=jax.ShapeDtypeStruct((1, 1), jnp.float32),
        out_specs=pl.BlockSpec(memory_space=pltpu.SMEM),
    )(psm3, rm3, anc, tmat, tgt)


def _f2i(x):
    return lax.bitcast_convert_type(x, jnp.int32)


def _i2f(x):
    return lax.bitcast_convert_type(x, jnp.float32)


def _sqrt(v):
    y = _i2f(jnp.int32(0x5F3759DF) - lax.shift_right_logical(_f2i(v), 1))
    for _ in range(3):
        y = y * (1.5 - 0.5 * v * y * y)
    return jnp.where(v <= 0.0, 0.0, v * y)


def _ln(x):
    bits = _f2i(x)
    e = lax.shift_right_logical(bits, 23) - 127
    m = _i2f(jnp.bitwise_or(jnp.bitwise_and(bits, 0x007FFFFF),
                            jnp.int32(0x3F800000)))
    big = m > 1.4142135
    m = jnp.where(big, m * 0.5, m)
    e = e + jnp.where(big, 1, 0)
    z = m - 1.0
    t = z / (2.0 + z)
    t2 = t * t
    p = 2.0 * t * (1.0 + t2 * (0.3333333333 + t2 * (0.2 + t2 * (0.14285714 + t2 * 0.11111111))))
    return e.astype(jnp.float32) * 0.6931471805599453 + p


def _sincos(x):
    t = x * 0.6366197723675814 + 12582912.0
    kf = t - 12582912.0
    q = kf.astype(jnp.int32)
    r = x - kf * 1.5707964
    r = r + kf * 4.371139e-08
    r2 = r * r
    sinp = r * (1.0 + r2 * (-0.16666667 + r2 * (0.008333333 + r2 * (-1.984127e-4))))
    cosp = 1.0 + r2 * (-0.5 + r2 * (0.041666668 + r2 * (-0.0013888889 + r2 * 2.4801587e-5)))
    qm = jnp.bitwise_and(q, 3)
    swap = jnp.bitwise_and(q, 1) == 1
    s_ = jnp.where(swap, cosp, sinp)
    c_ = jnp.where(swap, sinp, cosp)
    s_ = jnp.where(qm >= 2, -s_, s_)
    c_ = jnp.where(jnp.bitwise_and(qm + 1, 2) == 2, -c_, c_)
    return s_, c_


def _sc_body(data_hbm, tgt_hbm, tm_hbm, out_hbm,
             data_v, tgt_v, gx0_v, gx1_v, gy0_v, gy1_v, ga_v, tm_v, acc_v,
             *, chunk):
    cid = lax.axis_index("c")
    sid = lax.axis_index("s")
    wid = sid * 2 + cid
    pltpu.sync_copy(data_hbm.at[wid], data_v)
    pltpu.sync_copy(tgt_hbm, tgt_v)
    pltpu.sync_copy(tm_hbm, tm_v)

    tm = tm_v[...]
    r00, r01, r02, t0 = tm[0], tm[1], tm[2], tm[3]
    r10, r11, r12, t1 = tm[4], tm[5], tm[6], tm[7]

    for j in range(NTPAD // 16):
        sl = pl.ds(j * 16, 16)
        tx = tgt_v[0, sl]
        ty = tgt_v[1, sl]
        tw = tgt_v[4, sl]
        tl = tgt_v[5, sl]
        s_, c_ = _sincos(tgt_v[6, sl])
        ex = jnp.abs(c_) * tl * 0.5 + jnp.abs(s_) * tw * 0.5
        ey = jnp.abs(s_) * tl * 0.5 + jnp.abs(c_) * tw * 0.5
        x0 = tx - ex
        x1 = tx + ex
        y0 = ty - ey
        y1 = ty + ey
        gx0_v[sl] = x0
        gx1_v[sl] = x1
        gy0_v[sl] = y0
        gy1_v[sl] = y1
        ga_v[sl] = (x1 - x0) * (y1 - y0)

    def step(i, acc):
        boxes = []
        for g in range(GROUPS):
            sl = pl.ds(i * (16 * GROUPS) + g * 16, 16)
            lg = data_v[14, sl]
            prob = 1.0 / (1.0 + jnp.exp(-lg))
            wgt = jnp.where(prob > 0.1, _ln(1.0 - prob), 0.0)
            d0 = data_v[0, sl]
            d1 = data_v[1, sl]
            d2 = data_v[2, sl]
            d3 = data_v[3, sl]
            d4 = data_v[4, sl]
            d5 = data_v[5, sl]
            d6 = data_v[6, sl]
            a0 = data_v[7, sl]
            a1 = data_v[8, sl]
            a2 = data_v[9, sl]
            a3 = data_v[10, sl]
            a4 = data_v[11, sl]
            a5 = data_v[12, sl]
            a6 = data_v[13, sl]
            ad = _sqrt(a4 * a4 + a5 * a5)
            bx = d0 * ad + a0
            by = d1 * ad + a1
            bz = d2 * a3 + a2
            dh = jnp.exp(d3) * a3
            dw = jnp.exp(d4) * a4
            dl = jnp.exp(d5) * a5
            s_, c_ = _sincos(d6 + a6)
            cx = r00 * bx + r01 * by + r02 * bz + t0
            cy = r10 * bx + r11 * by + r12 * bz + t1
            ex = (jnp.abs(dl * 0.5 * (r00 * c_ + r01 * s_))
                  + jnp.abs(dw * 0.5 * (r01 * c_ - r00 * s_))
                  + jnp.abs(dh * 0.5 * r02))
            ey = (jnp.abs(dl * 0.5 * (r10 * c_ + r11 * s_))
                  + jnp.abs(dw * 0.5 * (r11 * c_ - r10 * s_))
                  + jnp.abs(dh * 0.5 * r12))
            px0 = cx - ex
            px1 = cx + ex
            py0 = cy - ey
            py1 = cy + ey
            pa = (px1 - px0) * (py1 - py0)
            boxes.append((px0, px1, py0, py1, pa, wgt))
        ssum = [jnp.zeros((16,), jnp.float32) for _ in range(GROUPS)]
        for j in range(NT // 16 + 1):
            tsl = pl.ds(j * 16, 16)
            g0 = gx0_v[tsl]
            g1 = gx1_v[tsl]
            h0 = gy0_v[tsl]
            h1 = gy1_v[tsl]
            gg = ga_v[tsl]
            for lane in range(16):
                n = j * 16 + lane
                if n >= NT:
                    break
                b0, b1, c0, c1, aa = g0[lane], g1[lane], h0[lane], h1[lane], gg[lane]
                for g in range(GROUPS):
                    px0, px1, py0, py1, pa, _ = boxes[g]
                    iw = jnp.maximum(0.0, jnp.minimum(px1, b1) - jnp.maximum(px0, b0))
                    ih = jnp.maximum(0.0, jnp.minimum(py1, c1) - jnp.maximum(py0, c0))
                    inter = iw * ih
                    ssum[g] = ssum[g] + inter / (pa + aa - inter)
        for g in range(GROUPS):
            acc = acc + boxes[g][5] * ssum[g]
        return acc

    acc = lax.fori_loop(0, chunk // 16 // GROUPS, step, jnp.zeros((16,), jnp.float32))
    acc_v[...] = acc
    pltpu.sync_copy(acc_v, out_hbm.at[wid])


def _sc_call(chunk):
    mesh = plsc.VectorSubcoreMesh(core_axis_name="c", subcore_axis_name="s")
    return functools.partial(
        pl.kernel,
        out_type=jax.ShapeDtypeStruct((NWORK, 16), jnp.float32),
        mesh=mesh,
        scratch_types=[
            pltpu.VMEM((15, chunk), jnp.float32),
            pltpu.VMEM((7, NTPAD), jnp.float32),
            pltpu.VMEM((NTPAD,), jnp.float32),
            pltpu.VMEM((NTPAD,), jnp.float32),
            pltpu.VMEM((NTPAD,), jnp.float32),
            pltpu.VMEM((NTPAD,), jnp.float32),
            pltpu.VMEM((NTPAD,), jnp.float32),
            pltpu.VMEM((16,), jnp.float32),
            pltpu.VMEM((16,), jnp.float32),
        ],
    )(functools.partial(_sc_body, chunk=chunk))


def kernel(psm, rm, anchor_box, transformation_matrix, target):
    f32 = jnp.float32
    tmat = transformation_matrix.astype(f32)
    tgt = jnp.transpose(target).astype(f32)

    npad = SC_PAD - SC_BOXES
    dpad = jnp.zeros((npad,), f32)
    apad = jnp.ones((npad,), f32)
    lpad = jnp.full((npad,), -100.0, f32)
    rows = []
    for c in range(7):
        rows.append(jnp.concatenate([rm[0, 7 + c].reshape(-1), dpad]))
    for c in range(7):
        rows.append(jnp.concatenate([anchor_box[:, :, 1, c].reshape(-1).astype(f32), apad]))
    rows.append(jnp.concatenate([psm[0, 1].reshape(-1), lpad]))
    data = jnp.stack(rows)
    data = data.reshape(15, NWORK, SC_CHUNK).transpose(1, 0, 2)
    tgt_pad = jnp.pad(tgt, ((0, 0), (0, NTPAD - NT)), constant_values=1.0)
    tm16 = jnp.concatenate([tmat[0], tmat[1], jnp.zeros((8,), f32)])
    sc_out = _sc_call(SC_CHUNK)(data, tgt_pad, tm16)

    anc0 = jnp.transpose(anchor_box[:, :, 0, :], (2, 0, 1)).astype(f32)
    tc_out = _tc_call(psm[0, 0:1], rm[0, 0:7], anc0, tmat, tgt)

    return tc_out[0, 0] + jnp.sum(sc_out)


# speedup vs baseline: 1.5045x; 1.0054x over previous
import functools

import jax
import jax.numpy as jnp
from jax import lax
from jax.experimental import pallas as pl
from jax.experimental.pallas import tpu as pltpu
from jax.experimental.pallas import tpu_sc as plsc

W, L, NA, NT = 100, 100, 2, 50
NWORK = 32
GROUPS = 4
NTPAD = 64
SC_BOXES = W * L
SC_PAD = 10240
SC_CORES = 1
SC_CHUNK = SC_PAD // (16 * SC_CORES)


def _tc_body(psm_ref, rm_ref, anc_ref, t_ref, tgt_ref, out_ref):
    def sc(i, j):
        return t_ref[i:i + 1, j:j + 1]

    r00, r01, r02, t0 = sc(0, 0), sc(0, 1), sc(0, 2), sc(0, 3)
    r10, r11, r12, t1 = sc(1, 0), sc(1, 1), sc(1, 2), sc(1, 3)

    tx = tgt_ref[0:1, :]
    ty = tgt_ref[1:2, :]
    tw = tgt_ref[4:5, :]
    tl = tgt_ref[5:6, :]
    tc_, ts_ = jnp.cos(tgt_ref[6:7, :]), jnp.sin(tgt_ref[6:7, :])
    tex = jnp.abs(tc_) * tl * 0.5 + jnp.abs(ts_) * tw * 0.5
    tey = jnp.abs(ts_) * tl * 0.5 + jnp.abs(tc_) * tw * 0.5
    gxmin, gxmax = tx - tex, tx + tex
    gymin, gymax = ty - tey, ty + tey
    garea = (gxmax - gxmin) * (gymax - gymin)

    total = jnp.float32(0.0)
    for a in range(psm_ref.shape[0]):
        lg = psm_ref[a]
        prob = jax.nn.sigmoid(lg)
        wgt = jnp.where(prob > 0.1, jnp.log(1.0 - prob), 0.0)

        d = [rm_ref[a * 7 + c] for c in range(7)]
        an = [anc_ref[a * 7 + c] for c in range(7)]
        ad = jnp.sqrt(an[4] * an[4] + an[5] * an[5])
        bx = d[0] * ad + an[0]
        by = d[1] * ad + an[1]
        bz = d[2] * an[3] + an[2]
        dh = jnp.exp(d[3]) * an[3]
        dw = jnp.exp(d[4]) * an[4]
        dl = jnp.exp(d[5]) * an[5]
        ry = d[6] + an[6]
        c, s = jnp.cos(ry), jnp.sin(ry)

        cx = r00 * bx + r01 * by + r02 * bz + t0
        cy = r10 * bx + r11 * by + r12 * bz + t1
        ex = (jnp.abs(dl * 0.5 * (r00 * c + r01 * s))
              + jnp.abs(dw * 0.5 * (r01 * c - r00 * s))
              + jnp.abs(dh * 0.5 * r02))
        ey = (jnp.abs(dl * 0.5 * (r10 * c + r11 * s))
              + jnp.abs(dw * 0.5 * (r11 * c - r10 * s))
              + jnp.abs(dh * 0.5 * r12))
        pxmin, pxmax = cx - ex, cx + ex
        pymin, pymax = cy - ey, cy + ey
        parea = (pxmax - pxmin) * (pymax - pymin)

        iousum = jnp.zeros((W, L), jnp.float32)
        for n in range(NT):
            xm = gxmin[0:1, n:n + 1]
            xM = gxmax[0:1, n:n + 1]
            ym = gymin[0:1, n:n + 1]
            yM = gymax[0:1, n:n + 1]
            ga = garea[0:1, n:n + 1]
            iw = jnp.maximum(0.0, jnp.minimum(pxmax, xM) - jnp.maximum(pxmin, xm))
            ih = jnp.maximum(0.0, jnp.minimum(pymax, yM) - jnp.maximum(pymin, ym))
            inter = iw * ih
            iousum = iousum + inter / (parea + ga - inter)
        total = total + jnp.sum(wgt * iousum)

    out_ref[0, 0] = total


def _tc_call(psm3, rm3, anc, tmat, tgt):
    return pl.pallas_call(
        _tc_body,
        out_shape=jax.ShapeDtypeStruct((1, 1), jnp.float32),
        out_specs=pl.BlockSpec(memory_space=pltpu.SMEM),
    )(psm3, rm3, anc, tmat, tgt)


def _f2i(x):
    return lax.bitcast_convert_type(x, jnp.int32)


def _i2f(x):
    return lax.bitcast_convert_type(x, jnp.float32)


def _sqrt(v):
    y = _i2f(jnp.int32(0x5F3759DF) - lax.shift_right_logical(_f2i(v), 1))
    for _ in range(3):
        y = y * (1.5 - 0.5 * v * y * y)
    return jnp.where(v <= 0.0, 0.0, v * y)


def _ln(x):
    bits = _f2i(x)
    e = lax.shift_right_logical(bits, 23) - 127
    m = _i2f(jnp.bitwise_or(jnp.bitwise_and(bits, 0x007FFFFF),
                            jnp.int32(0x3F800000)))
    big = m > 1.4142135
    m = jnp.where(big, m * 0.5, m)
    e = e + jnp.where(big, 1, 0)
    z = m - 1.0
    t = z / (2.0 + z)
    t2 = t * t
    p = 2.0 * t * (1.0 + t2 * (0.3333333333 + t2 * (0.2 + t2 * (0.14285714 + t2 * 0.11111111))))
    return e.astype(jnp.float32) * 0.6931471805599453 + p


def _sincos(x):
    t = x * 0.6366197723675814 + 12582912.0
    kf = t - 12582912.0
    q = kf.astype(jnp.int32)
    r = x - kf * 1.5707964
    r = r + kf * 4.371139e-08
    r2 = r * r
    sinp = r * (1.0 + r2 * (-0.16666667 + r2 * (0.008333333 + r2 * (-1.984127e-4))))
    cosp = 1.0 + r2 * (-0.5 + r2 * (0.041666668 + r2 * (-0.0013888889 + r2 * 2.4801587e-5)))
    qm = jnp.bitwise_and(q, 3)
    swap = jnp.bitwise_and(q, 1) == 1
    s_ = jnp.where(swap, cosp, sinp)
    c_ = jnp.where(swap, sinp, cosp)
    s_ = jnp.where(qm >= 2, -s_, s_)
    c_ = jnp.where(jnp.bitwise_and(qm + 1, 2) == 2, -c_, c_)
    return s_, c_


def _sc_body(data_hbm, tgt_hbm, tm_hbm, out_hbm,
             data_v, tgt_v, gx0_v, gx1_v, gy0_v, gy1_v, ga_v, tm_v, acc_v,
             *, chunk, num_cores):
    cid = lax.axis_index("c")
    sid = lax.axis_index("s")
    wid = sid * num_cores + cid
    pltpu.sync_copy(data_hbm.at[wid], data_v)
    pltpu.sync_copy(tgt_hbm, tgt_v)
    pltpu.sync_copy(tm_hbm, tm_v)

    tm = tm_v[...]
    r00, r01, r02, t0 = tm[0], tm[1], tm[2], tm[3]
    r10, r11, r12, t1 = tm[4], tm[5], tm[6], tm[7]

    for j in range(NTPAD // 16):
        sl = pl.ds(j * 16, 16)
        tx = tgt_v[0, sl]
        ty = tgt_v[1, sl]
        tw = tgt_v[4, sl]
        tl = tgt_v[5, sl]
        s_, c_ = _sincos(tgt_v[6, sl])
        ex = jnp.abs(c_) * tl * 0.5 + jnp.abs(s_) * tw * 0.5
        ey = jnp.abs(s_) * tl * 0.5 + jnp.abs(c_) * tw * 0.5
        x0 = tx - ex
        x1 = tx + ex
        y0 = ty - ey
        y1 = ty + ey
        gx0_v[sl] = x0
        gx1_v[sl] = x1
        gy0_v[sl] = y0
        gy1_v[sl] = y1
        ga_v[sl] = (x1 - x0) * (y1 - y0)

    def step(i, acc):
        boxes = []
        for g in range(GROUPS):
            sl = pl.ds(i * (16 * GROUPS) + g * 16, 16)
            lg = data_v[14, sl]
            prob = 1.0 / (1.0 + jnp.exp(-lg))
            wgt = jnp.where(prob > 0.1, _ln(1.0 - prob), 0.0)
            d0 = data_v[0, sl]
            d1 = data_v[1, sl]
            d2 = data_v[2, sl]
            d3 = data_v[3, sl]
            d4 = data_v[4, sl]
            d5 = data_v[5, sl]
            d6 = data_v[6, sl]
            a0 = data_v[7, sl]
            a1 = data_v[8, sl]
            a2 = data_v[9, sl]
            a3 = data_v[10, sl]
            a4 = data_v[11, sl]
            a5 = data_v[12, sl]
            a6 = data_v[13, sl]
            ad = _sqrt(a4 * a4 + a5 * a5)
            bx = d0 * ad + a0
            by = d1 * ad + a1
            bz = d2 * a3 + a2
            dh = jnp.exp(d3) * a3
            dw = jnp.exp(d4) * a4
            dl = jnp.exp(d5) * a5
            s_, c_ = _sincos(d6 + a6)
            cx = r00 * bx + r01 * by + r02 * bz + t0
            cy = r10 * bx + r11 * by + r12 * bz + t1
            ex = (jnp.abs(dl * 0.5 * (r00 * c_ + r01 * s_))
                  + jnp.abs(dw * 0.5 * (r01 * c_ - r00 * s_))
                  + jnp.abs(dh * 0.5 * r02))
            ey = (jnp.abs(dl * 0.5 * (r10 * c_ + r11 * s_))
                  + jnp.abs(dw * 0.5 * (r11 * c_ - r10 * s_))
                  + jnp.abs(dh * 0.5 * r12))
            px0 = cx - ex
            px1 = cx + ex
            py0 = cy - ey
            py1 = cy + ey
            pa = (px1 - px0) * (py1 - py0)
            boxes.append((px0, px1, py0, py1, pa, wgt))
        ssum = [jnp.zeros((16,), jnp.float32) for _ in range(GROUPS)]
        for j in range(NT // 16 + 1):
            tsl = pl.ds(j * 16, 16)
            g0 = gx0_v[tsl]
            g1 = gx1_v[tsl]
            h0 = gy0_v[tsl]
            h1 = gy1_v[tsl]
            gg = ga_v[tsl]
            for lane in range(16):
                n = j * 16 + lane
                if n >= NT:
                    break
                b0, b1, c0, c1, aa = g0[lane], g1[lane], h0[lane], h1[lane], gg[lane]
                for g in range(GROUPS):
                    px0, px1, py0, py1, pa, _ = boxes[g]
                    iw = jnp.maximum(0.0, jnp.minimum(px1, b1) - jnp.maximum(px0, b0))
                    ih = jnp.maximum(0.0, jnp.minimum(py1, c1) - jnp.maximum(py0, c0))
                    inter = iw * ih
                    ssum[g] = ssum[g] + inter / (pa + aa - inter)
        for g in range(GROUPS):
            acc = acc + boxes[g][5] * ssum[g]
        return acc

    acc = lax.fori_loop(0, chunk // 16 // GROUPS, step, jnp.zeros((16,), jnp.float32))
    acc_v[...] = acc
    pltpu.sync_copy(acc_v, out_hbm.at[wid])


def _sc_call(chunk, num_cores=2):
    mesh = plsc.VectorSubcoreMesh(core_axis_name="c", subcore_axis_name="s",
                                  num_cores=num_cores)
    nwork = num_cores * 16
    return functools.partial(
        pl.kernel,
        out_type=jax.ShapeDtypeStruct((nwork, 16), jnp.float32),
        mesh=mesh,
        scratch_types=[
            pltpu.VMEM((15, chunk), jnp.float32),
            pltpu.VMEM((7, NTPAD), jnp.float32),
            pltpu.VMEM((NTPAD,), jnp.float32),
            pltpu.VMEM((NTPAD,), jnp.float32),
            pltpu.VMEM((NTPAD,), jnp.float32),
            pltpu.VMEM((NTPAD,), jnp.float32),
            pltpu.VMEM((NTPAD,), jnp.float32),
            pltpu.VMEM((16,), jnp.float32),
            pltpu.VMEM((16,), jnp.float32),
        ],
    )(functools.partial(_sc_body, chunk=chunk, num_cores=num_cores))


def kernel(psm, rm, anchor_box, transformation_matrix, target):
    f32 = jnp.float32
    tmat = transformation_matrix.astype(f32)
    tgt = jnp.transpose(target).astype(f32)

    npad = SC_PAD - SC_BOXES
    dpad = jnp.zeros((npad,), f32)
    apad = jnp.ones((npad,), f32)
    lpad = jnp.full((npad,), -100.0, f32)
    rows = []
    for c in range(7):
        rows.append(jnp.concatenate([rm[0, 7 + c].reshape(-1), dpad]))
    for c in range(7):
        rows.append(jnp.concatenate([anchor_box[:, :, 1, c].reshape(-1).astype(f32), apad]))
    rows.append(jnp.concatenate([psm[0, 1].reshape(-1), lpad]))
    data = jnp.stack(rows)
    data = data.reshape(15, 16 * SC_CORES, SC_CHUNK).transpose(1, 0, 2)
    tgt_pad = jnp.pad(tgt, ((0, 0), (0, NTPAD - NT)), constant_values=1.0)
    tm16 = jnp.concatenate([tmat[0], tmat[1], jnp.zeros((8,), f32)])
    sc_out = _sc_call(SC_CHUNK, SC_CORES)(data, tgt_pad, tm16)

    anc0 = jnp.transpose(anchor_box[:, :, 0, :], (2, 0, 1)).astype(f32)
    tc_out = _tc_call(psm[0, 0:1], rm[0, 0:7], anc0, tmat, tgt)

    return tc_out[0, 0] + jnp.sum(sc_out)


# speedup vs baseline: 1.5352x; 1.0204x over previous
import functools

import jax
import jax.numpy as jnp
from jax import lax
from jax.experimental import pallas as pl
from jax.experimental.pallas import tpu as pltpu
from jax.experimental.pallas import tpu_sc as plsc

W, L, NA, NT = 100, 100, 2, 50
NWORK = 32
GROUPS = 2
NTPAD = 64
SC_BOXES = W * L
SC_PAD = 10240
SC_CORES = 1
SC_CHUNK = SC_PAD // (16 * SC_CORES)


def _tc_body(psm_ref, rm_ref, anc_ref, t_ref, tgt_ref, out_ref):
    def sc(i, j):
        return t_ref[i:i + 1, j:j + 1]

    r00, r01, r02, t0 = sc(0, 0), sc(0, 1), sc(0, 2), sc(0, 3)
    r10, r11, r12, t1 = sc(1, 0), sc(1, 1), sc(1, 2), sc(1, 3)

    tx = tgt_ref[0:1, :]
    ty = tgt_ref[1:2, :]
    tw = tgt_ref[4:5, :]
    tl = tgt_ref[5:6, :]
    tc_, ts_ = jnp.cos(tgt_ref[6:7, :]), jnp.sin(tgt_ref[6:7, :])
    tex = jnp.abs(tc_) * tl * 0.5 + jnp.abs(ts_) * tw * 0.5
    tey = jnp.abs(ts_) * tl * 0.5 + jnp.abs(tc_) * tw * 0.5
    gxmin, gxmax = tx - tex, tx + tex
    gymin, gymax = ty - tey, ty + tey
    garea = (gxmax - gxmin) * (gymax - gymin)

    total = jnp.float32(0.0)
    for a in range(psm_ref.shape[0]):
        lg = psm_ref[a]
        prob = jax.nn.sigmoid(lg)
        wgt = jnp.where(prob > 0.1, jnp.log(1.0 - prob), 0.0)

        d = [rm_ref[a * 7 + c] for c in range(7)]
        an = [anc_ref[a * 7 + c] for c in range(7)]
        ad = jnp.sqrt(an[4] * an[4] + an[5] * an[5])
        bx = d[0] * ad + an[0]
        by = d[1] * ad + an[1]
        bz = d[2] * an[3] + an[2]
        dh = jnp.exp(d[3]) * an[3]
        dw = jnp.exp(d[4]) * an[4]
        dl = jnp.exp(d[5]) * an[5]
        ry = d[6] + an[6]
        c, s = jnp.cos(ry), jnp.sin(ry)

        cx = r00 * bx + r01 * by + r02 * bz + t0
        cy = r10 * bx + r11 * by + r12 * bz + t1
        ex = (jnp.abs(dl * 0.5 * (r00 * c + r01 * s))
              + jnp.abs(dw * 0.5 * (r01 * c - r00 * s))
              + jnp.abs(dh * 0.5 * r02))
        ey = (jnp.abs(dl * 0.5 * (r10 * c + r11 * s))
              + jnp.abs(dw * 0.5 * (r11 * c - r10 * s))
              + jnp.abs(dh * 0.5 * r12))
        pxmin, pxmax = cx - ex, cx + ex
        pymin, pymax = cy - ey, cy + ey
        parea = (pxmax - pxmin) * (pymax - pymin)

        iousum = jnp.zeros((W, L), jnp.float32)
        for n in range(NT):
            xm = gxmin[0:1, n:n + 1]
            xM = gxmax[0:1, n:n + 1]
            ym = gymin[0:1, n:n + 1]
            yM = gymax[0:1, n:n + 1]
            ga = garea[0:1, n:n + 1]
            iw = jnp.maximum(0.0, jnp.minimum(pxmax, xM) - jnp.maximum(pxmin, xm))
            ih = jnp.maximum(0.0, jnp.minimum(pymax, yM) - jnp.maximum(pymin, ym))
            inter = iw * ih
            iousum = iousum + inter / (parea + ga - inter)
        total = total + jnp.sum(wgt * iousum)

    out_ref[0, 0] = total


def _tc_call(psm3, rm3, anc, tmat, tgt):
    return pl.pallas_call(
        _tc_body,
        out_shape=jax.ShapeDtypeStruct((1, 1), jnp.float32),
        out_specs=pl.BlockSpec(memory_space=pltpu.SMEM),
    )(psm3, rm3, anc, tmat, tgt)


def _f2i(x):
    return lax.bitcast_convert_type(x, jnp.int32)


def _i2f(x):
    return lax.bitcast_convert_type(x, jnp.float32)


def _sqrt(v):
    y = _i2f(jnp.int32(0x5F3759DF) - lax.shift_right_logical(_f2i(v), 1))
    for _ in range(3):
        y = y * (1.5 - 0.5 * v * y * y)
    return jnp.where(v <= 0.0, 0.0, v * y)


def _ln(x):
    bits = _f2i(x)
    e = lax.shift_right_logical(bits, 23) - 127
    m = _i2f(jnp.bitwise_or(jnp.bitwise_and(bits, 0x007FFFFF),
                            jnp.int32(0x3F800000)))
    big = m > 1.4142135
    m = jnp.where(big, m * 0.5, m)
    e = e + jnp.where(big, 1, 0)
    z = m - 1.0
    t = z / (2.0 + z)
    t2 = t * t
    p = 2.0 * t * (1.0 + t2 * (0.3333333333 + t2 * (0.2 + t2 * (0.14285714 + t2 * 0.11111111))))
    return e.astype(jnp.float32) * 0.6931471805599453 + p


def _sincos(x):
    t = x * 0.6366197723675814 + 12582912.0
    kf = t - 12582912.0
    q = kf.astype(jnp.int32)
    r = x - kf * 1.5707964
    r = r + kf * 4.371139e-08
    r2 = r * r
    sinp = r * (1.0 + r2 * (-0.16666667 + r2 * (0.008333333 + r2 * (-1.984127e-4))))
    cosp = 1.0 + r2 * (-0.5 + r2 * (0.041666668 + r2 * (-0.0013888889 + r2 * 2.4801587e-5)))
    qm = jnp.bitwise_and(q, 3)
    swap = jnp.bitwise_and(q, 1) == 1
    s_ = jnp.where(swap, cosp, sinp)
    c_ = jnp.where(swap, sinp, cosp)
    s_ = jnp.where(qm >= 2, -s_, s_)
    c_ = jnp.where(jnp.bitwise_and(qm + 1, 2) == 2, -c_, c_)
    return s_, c_


def _sc_body(data_hbm, tgt_hbm, tm_hbm, out_hbm,
             data_v, tgt_v, gx0_v, gx1_v, gy0_v, gy1_v, ga_v, tm_v, acc_v,
             *, chunk, num_cores):
    cid = lax.axis_index("c")
    sid = lax.axis_index("s")
    wid = sid * num_cores + cid
    pltpu.sync_copy(data_hbm.at[wid], data_v)
    pltpu.sync_copy(tgt_hbm, tgt_v)
    pltpu.sync_copy(tm_hbm, tm_v)

    tm = tm_v[...]
    r00, r01, r02, t0 = tm[0], tm[1], tm[2], tm[3]
    r10, r11, r12, t1 = tm[4], tm[5], tm[6], tm[7]

    for j in range(NTPAD // 16):
        sl = pl.ds(j * 16, 16)
        tx = tgt_v[0, sl]
        ty = tgt_v[1, sl]
        tw = tgt_v[4, sl]
        tl = tgt_v[5, sl]
        s_, c_ = _sincos(tgt_v[6, sl])
        ex = jnp.abs(c_) * tl * 0.5 + jnp.abs(s_) * tw * 0.5
        ey = jnp.abs(s_) * tl * 0.5 + jnp.abs(c_) * tw * 0.5
        x0 = tx - ex
        x1 = tx + ex
        y0 = ty - ey
        y1 = ty + ey
        gx0_v[sl] = x0
        gx1_v[sl] = x1
        gy0_v[sl] = y0
        gy1_v[sl] = y1
        ga_v[sl] = (x1 - x0) * (y1 - y0)

    def step(i, acc):
        boxes = []
        for g in range(GROUPS):
            sl = pl.ds(i * (16 * GROUPS) + g * 16, 16)
            lg = data_v[14, sl]
            prob = 1.0 / (1.0 + jnp.exp(-lg))
            wgt = jnp.where(prob > 0.1, _ln(1.0 - prob), 0.0)
            d0 = data_v[0, sl]
            d1 = data_v[1, sl]
            d2 = data_v[2, sl]
            d3 = data_v[3, sl]
            d4 = data_v[4, sl]
            d5 = data_v[5, sl]
            d6 = data_v[6, sl]
            a0 = data_v[7, sl]
            a1 = data_v[8, sl]
            a2 = data_v[9, sl]
            a3 = data_v[10, sl]
            a4 = data_v[11, sl]
            a5 = data_v[12, sl]
            a6 = data_v[13, sl]
            ad = _sqrt(a4 * a4 + a5 * a5)
            bx = d0 * ad + a0
            by = d1 * ad + a1
            bz = d2 * a3 + a2
            dh = jnp.exp(d3) * a3
            dw = jnp.exp(d4) * a4
            dl = jnp.exp(d5) * a5
            s_, c_ = _sincos(d6 + a6)
            cx = r00 * bx + r01 * by + r02 * bz + t0
            cy = r10 * bx + r11 * by + r12 * bz + t1
            ex = (jnp.abs(dl * 0.5 * (r00 * c_ + r01 * s_))
                  + jnp.abs(dw * 0.5 * (r01 * c_ - r00 * s_))
                  + jnp.abs(dh * 0.5 * r02))
            ey = (jnp.abs(dl * 0.5 * (r10 * c_ + r11 * s_))
                  + jnp.abs(dw * 0.5 * (r11 * c_ - r10 * s_))
                  + jnp.abs(dh * 0.5 * r12))
            px0 = cx - ex
            px1 = cx + ex
            py0 = cy - ey
            py1 = cy + ey
            pa = (px1 - px0) * (py1 - py0)
            boxes.append((px0, px1, py0, py1, pa, wgt))
        ssum = [jnp.zeros((16,), jnp.float32) for _ in range(GROUPS)]
        for j in range(NT // 16 + 1):
            tsl = pl.ds(j * 16, 16)
            g0 = gx0_v[tsl]
            g1 = gx1_v[tsl]
            h0 = gy0_v[tsl]
            h1 = gy1_v[tsl]
            gg = ga_v[tsl]
            for lane in range(16):
                n = j * 16 + lane
                if n >= NT:
                    break
                b0, b1, c0, c1, aa = g0[lane], g1[lane], h0[lane], h1[lane], gg[lane]
                for g in range(GROUPS):
                    px0, px1, py0, py1, pa, _ = boxes[g]
                    iw = jnp.maximum(0.0, jnp.minimum(px1, b1) - jnp.maximum(px0, b0))
                    ih = jnp.maximum(0.0, jnp.minimum(py1, c1) - jnp.maximum(py0, c0))
                    inter = iw * ih
                    ssum[g] = ssum[g] + inter / (pa + aa - inter)
        for g in range(GROUPS):
            acc = acc + boxes[g][5] * ssum[g]
        return acc

    acc = lax.fori_loop(0, chunk // 16 // GROUPS, step, jnp.zeros((16,), jnp.float32))
    acc_v[...] = acc
    pltpu.sync_copy(acc_v, out_hbm.at[wid])


def _sc_call(chunk, num_cores=2):
    mesh = plsc.VectorSubcoreMesh(core_axis_name="c", subcore_axis_name="s",
                                  num_cores=num_cores)
    nwork = num_cores * 16
    return functools.partial(
        pl.kernel,
        out_type=jax.ShapeDtypeStruct((nwork, 16), jnp.float32),
        mesh=mesh,
        scratch_types=[
            pltpu.VMEM((15, chunk), jnp.float32),
            pltpu.VMEM((7, NTPAD), jnp.float32),
            pltpu.VMEM((NTPAD,), jnp.float32),
            pltpu.VMEM((NTPAD,), jnp.float32),
            pltpu.VMEM((NTPAD,), jnp.float32),
            pltpu.VMEM((NTPAD,), jnp.float32),
            pltpu.VMEM((NTPAD,), jnp.float32),
            pltpu.VMEM((16,), jnp.float32),
            pltpu.VMEM((16,), jnp.float32),
        ],
    )(functools.partial(_sc_body, chunk=chunk, num_cores=num_cores))


def kernel(psm, rm, anchor_box, transformation_matrix, target):
    f32 = jnp.float32
    tmat = transformation_matrix.astype(f32)
    tgt = jnp.transpose(target).astype(f32)

    npad = SC_PAD - SC_BOXES
    dpad = jnp.zeros((npad,), f32)
    apad = jnp.ones((npad,), f32)
    lpad = jnp.full((npad,), -100.0, f32)
    rows = []
    for c in range(7):
        rows.append(jnp.concatenate([rm[0, 7 + c].reshape(-1), dpad]))
    for c in range(7):
        rows.append(jnp.concatenate([anchor_box[:, :, 1, c].reshape(-1).astype(f32), apad]))
    rows.append(jnp.concatenate([psm[0, 1].reshape(-1), lpad]))
    data = jnp.stack(rows)
    data = data.reshape(15, 16 * SC_CORES, SC_CHUNK).transpose(1, 0, 2)
    tgt_pad = jnp.pad(tgt, ((0, 0), (0, NTPAD - NT)), constant_values=1.0)
    tm16 = jnp.concatenate([tmat[0], tmat[1], jnp.zeros((8,), f32)])
    sc_out = _sc_call(SC_CHUNK, SC_CORES)(data, tgt_pad, tm16)

    anc0 = jnp.transpose(anchor_box[:, :, 0, :], (2, 0, 1)).astype(f32)
    tc_out = _tc_call(psm[0, 0:1], rm[0, 0:7], anc0, tmat, tgt)

    return tc_out[0, 0] + jnp.sum(sc_out)
